# baseline (speedup 1.0000x reference)
import functools

import jax
import jax.numpy as jnp
from jax import lax
from jax.experimental import pallas as pl
from jax.experimental.pallas import tpu as pltpu
from jax.experimental.pallas import tpu_sc as plsc

_LANES = 16


def kernel(inputs, relative_embedding):
    seq_len = inputs.shape[1]
    num_rows, d = relative_embedding.shape
    max_d = (num_rows - 1) // 2
    lo, hi = -seq_len + 1, seq_len - 1

    n_pad = ((num_rows + _LANES - 1) // _LANES) * _LANES

    mesh = plsc.VectorSubcoreMesh(core_axis_name="c", subcore_axis_name="s")

    @functools.partial(
        pl.kernel,
        mesh=mesh,
        out_type=jax.ShapeDtypeStruct((num_rows, d), jnp.float32),
        scratch_types=[
            pltpu.VMEM((n_pad,), jnp.int32),
            pltpu.VMEM((n_pad, d), jnp.float32),
            pltpu.SemaphoreType.DMA,
        ],
        compiler_params=pltpu.CompilerParams(use_tc_tiling_on_sc=False),
    )
    def emb_gather(table_hbm, out_hbm, idx_v, rows_v, sem):
        cid = lax.axis_index("c")
        sid = lax.axis_index("s")

        shift = n_pad - num_rows
        @pl.when(jnp.logical_and(cid == 0, sid == 0))
        def _():
            for k in range(n_pad // _LANES):
                p = lax.iota(jnp.int32, _LANES) + (k * _LANES)
                o = p - jnp.where(p >= num_rows - 1, shift, 0)
                r = jnp.minimum(jnp.maximum(o - max_d, lo), hi) + max_d
                idx_v[pl.ds(k * _LANES, _LANES)] = r
            pltpu.async_copy(table_hbm.at[idx_v], rows_v, sem).wait()
            cp1 = pltpu.async_copy(
                rows_v.at[pl.ds(0, num_rows - 1)],
                out_hbm.at[pl.ds(0, num_rows - 1)],
                sem,
            )
            cp2 = pltpu.async_copy(
                rows_v.at[pl.ds(n_pad - 8, 8)],
                out_hbm.at[pl.ds(num_rows - 8, 8)],
                sem,
            )
            cp1.wait()
            cp2.wait()

    return emb_gather(relative_embedding)


# speedup vs baseline: 1.1564x; 1.1564x over previous
import functools

import jax
import jax.numpy as jnp
from jax import lax
from jax.experimental import pallas as pl
from jax.experimental.pallas import tpu as pltpu
from jax.experimental.pallas import tpu_sc as plsc

_LANES = 16


def kernel(inputs, relative_embedding):
    seq_len = inputs.shape[1]
    num_rows, d = relative_embedding.shape
    max_d = (num_rows - 1) // 2
    lo, hi = -seq_len + 1, seq_len - 1

    info = plsc.get_sparse_core_info()
    nw = info.num_cores * info.num_subcores
    n2 = num_rows - nw if num_rows > nw else 0

    mesh = plsc.VectorSubcoreMesh(core_axis_name="c", subcore_axis_name="s")

    @functools.partial(
        pl.kernel,
        mesh=mesh,
        out_type=jax.ShapeDtypeStruct((num_rows, d), jnp.float32),
        scratch_types=[
            pltpu.VMEM((_LANES,), jnp.int32),
            pltpu.VMEM((2, d), jnp.float32),
            pltpu.SemaphoreType.DMA,
        ],
        compiler_params=pltpu.CompilerParams(use_tc_tiling_on_sc=False),
    )
    def emb_gather(table_hbm, out_hbm, idx_v, rows_v, sem):
        cid = lax.axis_index("c")
        sid = lax.axis_index("s")
        wid = sid * info.num_cores + cid

        base = jnp.where(wid < n2, 2 * wid, wid + n2)

        p = lax.iota(jnp.int32, _LANES) + base
        r = jnp.minimum(jnp.maximum(p - max_d, lo), hi) + max_d
        idx_v[...] = jnp.minimum(r, num_rows - 1)

        @pl.when(wid < n2)
        def _two_rows():
            pltpu.async_copy(
                table_hbm.at[idx_v.at[pl.ds(0, 2)]], rows_v, sem
            ).wait()
            pltpu.sync_copy(rows_v, out_hbm.at[pl.ds(base, 2)])

        @pl.when(jnp.logical_and(wid >= n2, base < num_rows))
        def _one_row():
            pltpu.async_copy(
                table_hbm.at[idx_v.at[pl.ds(0, 1)]],
                rows_v.at[pl.ds(0, 1)],
                sem,
            ).wait()
            pltpu.sync_copy(rows_v.at[pl.ds(0, 1)], out_hbm.at[pl.ds(base, 1)])

    return emb_gather(relative_embedding)


# speedup vs baseline: 1.2237x; 1.0582x over previous
import functools

import jax
import jax.numpy as jnp
from jax import lax
from jax.experimental import pallas as pl
from jax.experimental.pallas import tpu as pltpu
from jax.experimental.pallas import tpu_sc as plsc

_LANES = 16
_NUM_CORES = 1


def kernel(inputs, relative_embedding):
    seq_len = inputs.shape[1]
    num_rows, d = relative_embedding.shape
    max_d = (num_rows - 1) // 2
    lo, hi = -seq_len + 1, seq_len - 1

    info = plsc.get_sparse_core_info()
    nw = _NUM_CORES * info.num_subcores
    b = -(-num_rows // nw)
    n_big = num_rows - (b - 1) * nw

    mesh = plsc.VectorSubcoreMesh(
        core_axis_name="c", subcore_axis_name="s", num_cores=_NUM_CORES
    )

    @functools.partial(
        pl.kernel,
        mesh=mesh,
        out_type=jax.ShapeDtypeStruct((num_rows, d), jnp.float32),
        scratch_types=[
            pltpu.VMEM((_LANES,), jnp.int32),
            pltpu.VMEM((b, d), jnp.float32),
            pltpu.SemaphoreType.DMA,
        ],
        compiler_params=pltpu.CompilerParams(use_tc_tiling_on_sc=False),
    )
    def emb_gather(table_hbm, out_hbm, idx_v, rows_v, sem):
        cid = lax.axis_index("c")
        sid = lax.axis_index("s")
        wid = sid * _NUM_CORES + cid

        base = jnp.where(wid < n_big, b * wid, (b - 1) * wid + n_big)

        p = lax.iota(jnp.int32, _LANES) + base
        r = jnp.minimum(jnp.maximum(p - max_d, lo), hi) + max_d
        idx_v[...] = jnp.minimum(r, num_rows - 1)

        @pl.when(wid < n_big)
        def _big():
            pltpu.async_copy(
                table_hbm.at[idx_v.at[pl.ds(0, b)]], rows_v, sem
            ).wait()
            pltpu.sync_copy(rows_v, out_hbm.at[pl.ds(base, b)])

        if b > 1:

            @pl.when(wid >= n_big)
            def _small():
                pltpu.async_copy(
                    table_hbm.at[idx_v.at[pl.ds(0, b - 1)]],
                    rows_v.at[pl.ds(0, b - 1)],
                    sem,
                ).wait()
                pltpu.sync_copy(
                    rows_v.at[pl.ds(0, b - 1)], out_hbm.at[pl.ds(base, b - 1)]
                )

    return emb_gather(relative_embedding)
